# baseline (speedup 1.0000x reference)
import functools

import jax
import jax.numpy as jnp
from jax.experimental import pallas as pl
from jax.experimental.pallas import tpu as pltpu


def _fused_kernel(x_ref, wf_ref, p_ref, mf_ref, mh_ref, ones_ref, o_ref,
                  y_scr, s_scr, *,
                  W, C_in, C, DB, HW, G, NL, ND, inv_cnt, eps):
    i = pl.program_id(0)
    j = pl.program_id(1)
    d = pl.program_id(2)
    L = DB * HW
    DHW = ND * L

    @pl.when(j < NL)
    def _conv_phase():
        p = j % 2
        @pl.when(d == 0)
        def _zero_stats():
            s_scr[p, :, 0:2] = jnp.zeros((C, 2), jnp.float32)

        xb = x_ref[...]
        st = jnp.concatenate(
            [jnp.roll(xb * mh_ref[0, 0:1, :], W, axis=1), xb,
             jnp.roll(xb * mh_ref[1, 0:1, :], -W, axis=1)],
            axis=0)
        y3 = jnp.dot(wf_ref[...], st, preferred_element_type=jnp.float32)
        y = (y3[C:2 * C] + jnp.roll(y3[0:C] * mf_ref[0, 0:1, :], 1, axis=1)) + (
            jnp.roll(y3[2 * C:] * mf_ref[1, 0:1, :], -1, axis=1)
            + p_ref[NL * i + j, :, 0:1])

        ones = ones_ref[...]
        s_scr[p, :, 0:1] = s_scr[p, :, 0:1] + jnp.dot(
            y, ones, preferred_element_type=jnp.float32)
        s_scr[p, :, 1:2] = s_scr[p, :, 1:2] + jnp.dot(
            y * y, ones, preferred_element_type=jnp.float32)

        y_scr[p, :, pl.ds(d * L, L)] = y.astype(jnp.bfloat16)

    @pl.when(j > 0)
    def _apply_phase():
        q = (j + 1) % 2
        n_a = NL * i + j - 1
        @pl.when(d == 0)
        def _finalize():
            ssum = s_scr[q, :, 0:1]
            ssq = s_scr[q, :, 1:2]
            cper = C // G
            ri = jax.lax.broadcasted_iota(jnp.int32, (C, C), 0) // cper
            ci = jax.lax.broadcasted_iota(jnp.int32, (C, C), 1) // cper
            onehot = (ri == ci).astype(jnp.float32)
            gsum = jnp.dot(onehot, ssum, precision=jax.lax.Precision.HIGHEST)
            gsq = jnp.dot(onehot, ssq, precision=jax.lax.Precision.HIGHEST)
            mean = gsum * inv_cnt
            var = jnp.maximum(gsq * inv_cnt - mean * mean, 0.0)
            rstd = jax.lax.rsqrt(var + eps)
            a = rstd * p_ref[n_a, :, 1:2]
            s_scr[q, :, 2:3] = a
            s_scr[q, :, 3:4] = p_ref[n_a, :, 2:3] - mean * a

        a = s_scr[q, :, 2:3]
        b = s_scr[q, :, 3:4]
        z = y_scr[q, :, pl.ds(d * L, L)].astype(jnp.float32) * a + b
        o_ref[...] = z * jax.nn.sigmoid(z)


def kernel(x, w, bias, gamma, beta, scale, shift):
    N, C_in, D, H, W = x.shape
    C = w.shape[-1]
    HW = H * W
    DHW = D * HW
    G = 8
    eps = 1e-5
    inv_cnt = 1.0 / (DHW * (C // G))

    xr = x.reshape(N, C_in, DHW).astype(jnp.bfloat16)
    wf = jnp.transpose(w, (1, 3, 0, 2)).reshape(3 * C, 3 * C_in)
    wf = wf.astype(jnp.bfloat16)
    sc1 = scale + 1.0
    g1 = gamma[None, :] * sc1
    g2 = beta[None, :] * sc1 + shift
    p_all = jnp.stack(
        [jnp.broadcast_to(bias.reshape(1, C), (N, C)), g1, g2], axis=-1)

    DB = 4
    L = DB * HW
    ND = D // DB
    CORES = 2
    NL = N // CORES
    lane = jnp.arange(L, dtype=jnp.int32)
    m31 = (lane % W != W - 1).astype(jnp.float32)
    m0 = (lane % W != 0).astype(jnp.float32)
    mf = jnp.broadcast_to(jnp.stack([m31, m0])[:, None, :], (2, 8, L))
    hrow = (lane // W) % H
    mh31 = (hrow != H - 1).astype(jnp.bfloat16)
    mh0 = (hrow != 0).astype(jnp.bfloat16)
    mh = jnp.broadcast_to(jnp.stack([mh31, mh0])[:, None, :], (2, 8, L))
    ones_col = jnp.ones((L, 1), jnp.float32)

    grid = (CORES, NL + 1, ND)
    kern = functools.partial(_fused_kernel, W=W, C_in=C_in, C=C, DB=DB,
                             HW=HW, G=G, NL=NL, ND=ND, inv_cnt=inv_cnt,
                             eps=eps)
    out = pl.pallas_call(
        kern,
        out_shape=jax.ShapeDtypeStruct((N, C, DHW), jnp.float32),
        grid=grid,
        in_specs=[
            pl.BlockSpec(
                (None, C_in, L),
                lambda i, j, d: (NL * i + jnp.minimum(j, NL - 1), 0,
                                 jnp.where(j == NL, ND - 1, d))),
            pl.BlockSpec((3 * C, 3 * C_in), lambda i, j, d: (0, 0)),
            pl.BlockSpec((N, C, 3), lambda i, j, d: (0, 0, 0)),
            pl.BlockSpec((2, 8, L), lambda i, j, d: (0, 0, 0)),
            pl.BlockSpec((2, 8, L), lambda i, j, d: (0, 0, 0)),
            pl.BlockSpec((L, 1), lambda i, j, d: (0, 0)),
        ],
        out_specs=pl.BlockSpec(
            (None, C, L),
            lambda i, j, d: (NL * i + jnp.maximum(j, 1) - 1, 0, d)),
        scratch_shapes=[
            pltpu.VMEM((2, C, DHW), jnp.bfloat16),
            pltpu.VMEM((2, C, 128), jnp.float32),
        ],
        compiler_params=pltpu.CompilerParams(
            dimension_semantics=("parallel", "arbitrary", "arbitrary")),
    )(xr, wf, p_all, mf, mh, ones_col)
    return out.reshape(N, C, D, H, W)


# speedup vs baseline: 1.0681x; 1.0681x over previous
import functools

import jax
import jax.numpy as jnp
from jax.experimental import pallas as pl
from jax.experimental.pallas import tpu as pltpu


def _fused_kernel(x_ref, wf_ref, p_ref, mf_ref, mh_ref, ones_ref, o_ref,
                  y_scr, s_scr, *,
                  W, C_in, C, DB, HW, G, NL, ND, inv_cnt, eps):
    i = pl.program_id(0)
    j = pl.program_id(1)
    d = pl.program_id(2)
    L = DB * HW
    DHW = ND * L

    @pl.when(j < NL)
    def _conv_phase():
        p = j % 2
        @pl.when(d == 0)
        def _zero_stats():
            s_scr[p, :, 0:2] = jnp.zeros((C, 2), jnp.float32)

        xb = x_ref[...].astype(jnp.bfloat16)
        st = jnp.concatenate(
            [jnp.roll(xb * mh_ref[0, 0:1, :], W, axis=1), xb,
             jnp.roll(xb * mh_ref[1, 0:1, :], -W, axis=1)],
            axis=0)
        y3 = jnp.dot(wf_ref[...], st, preferred_element_type=jnp.float32)
        y = (y3[C:2 * C] + jnp.roll(y3[0:C] * mf_ref[0, 0:1, :], 1, axis=1)) + (
            jnp.roll(y3[2 * C:] * mf_ref[1, 0:1, :], -1, axis=1)
            + p_ref[NL * i + j, :, 0:1])

        ones = ones_ref[...]
        s_scr[p, :, 0:1] = s_scr[p, :, 0:1] + jnp.dot(
            y, ones, preferred_element_type=jnp.float32)
        s_scr[p, :, 1:2] = s_scr[p, :, 1:2] + jnp.dot(
            y * y, ones, preferred_element_type=jnp.float32)

        y_scr[p, :, pl.ds(d * L, L)] = y.astype(jnp.bfloat16)

    @pl.when(j > 0)
    def _apply_phase():
        q = (j + 1) % 2
        n_a = NL * i + j - 1
        @pl.when(d == 0)
        def _finalize():
            ssum = s_scr[q, :, 0:1]
            ssq = s_scr[q, :, 1:2]
            cper = C // G
            ri = jax.lax.broadcasted_iota(jnp.int32, (C, C), 0) // cper
            ci = jax.lax.broadcasted_iota(jnp.int32, (C, C), 1) // cper
            onehot = (ri == ci).astype(jnp.float32)
            gsum = jnp.dot(onehot, ssum, precision=jax.lax.Precision.HIGHEST)
            gsq = jnp.dot(onehot, ssq, precision=jax.lax.Precision.HIGHEST)
            mean = gsum * inv_cnt
            var = jnp.maximum(gsq * inv_cnt - mean * mean, 0.0)
            rstd = jax.lax.rsqrt(var + eps)
            a = rstd * p_ref[n_a, :, 1:2]
            s_scr[q, :, 2:3] = a
            s_scr[q, :, 3:4] = p_ref[n_a, :, 2:3] - mean * a

        a = s_scr[q, :, 2:3]
        b = s_scr[q, :, 3:4]
        z = y_scr[q, :, pl.ds(d * L, L)].astype(jnp.float32) * a + b
        o_ref[...] = z * jax.nn.sigmoid(z)


def kernel(x, w, bias, gamma, beta, scale, shift):
    N, C_in, D, H, W = x.shape
    C = w.shape[-1]
    HW = H * W
    DHW = D * HW
    G = 8
    eps = 1e-5
    inv_cnt = 1.0 / (DHW * (C // G))

    xr = x.reshape(N, C_in, DHW)
    wf = jnp.transpose(w, (1, 3, 0, 2)).reshape(3 * C, 3 * C_in)
    wf = wf.astype(jnp.bfloat16)
    sc1 = scale + 1.0
    g1 = gamma[None, :] * sc1
    g2 = beta[None, :] * sc1 + shift
    p_all = jnp.stack(
        [jnp.broadcast_to(bias.reshape(1, C), (N, C)), g1, g2], axis=-1)

    DB = 4
    L = DB * HW
    ND = D // DB
    CORES = 2
    NL = N // CORES
    lane = jnp.arange(L, dtype=jnp.int32)
    m31 = (lane % W != W - 1).astype(jnp.float32)
    m0 = (lane % W != 0).astype(jnp.float32)
    mf = jnp.broadcast_to(jnp.stack([m31, m0])[:, None, :], (2, 8, L))
    hrow = (lane // W) % H
    mh31 = (hrow != H - 1).astype(jnp.bfloat16)
    mh0 = (hrow != 0).astype(jnp.bfloat16)
    mh = jnp.broadcast_to(jnp.stack([mh31, mh0])[:, None, :], (2, 8, L))
    ones_col = jnp.ones((L, 1), jnp.float32)

    grid = (CORES, NL + 1, ND)
    kern = functools.partial(_fused_kernel, W=W, C_in=C_in, C=C, DB=DB,
                             HW=HW, G=G, NL=NL, ND=ND, inv_cnt=inv_cnt,
                             eps=eps)
    out = pl.pallas_call(
        kern,
        out_shape=jax.ShapeDtypeStruct((N, C, DHW), jnp.float32),
        grid=grid,
        in_specs=[
            pl.BlockSpec(
                (None, C_in, L),
                lambda i, j, d: (NL * i + jnp.minimum(j, NL - 1), 0,
                                 jnp.where(j == NL, ND - 1, d))),
            pl.BlockSpec((3 * C, 3 * C_in), lambda i, j, d: (0, 0)),
            pl.BlockSpec((N, C, 3), lambda i, j, d: (0, 0, 0)),
            pl.BlockSpec((2, 8, L), lambda i, j, d: (0, 0, 0)),
            pl.BlockSpec((2, 8, L), lambda i, j, d: (0, 0, 0)),
            pl.BlockSpec((L, 1), lambda i, j, d: (0, 0)),
        ],
        out_specs=pl.BlockSpec(
            (None, C, L),
            lambda i, j, d: (NL * i + jnp.maximum(j, 1) - 1, 0, d)),
        scratch_shapes=[
            pltpu.VMEM((2, C, DHW), jnp.bfloat16),
            pltpu.VMEM((2, C, 128), jnp.float32),
        ],
        compiler_params=pltpu.CompilerParams(
            dimension_semantics=("parallel", "arbitrary", "arbitrary")),
    )(xr, wf, p_all, mf, mh, ones_col)
    return out.reshape(N, C, D, H, W)


# speedup vs baseline: 1.0742x; 1.0058x over previous
import functools

import jax
import jax.numpy as jnp
from jax.experimental import pallas as pl
from jax.experimental.pallas import tpu as pltpu


def _fused_kernel(x_ref, wf_ref, p_ref, mf_ref, mh_ref, ones_ref, o_ref,
                  y_scr, s_scr, *,
                  W, C_in, C, DB, HW, G, NL, ND, inv_cnt, eps):
    i = pl.program_id(0)
    j = pl.program_id(1)
    d = pl.program_id(2)
    L = DB * HW
    DHW = ND * L

    @pl.when(j < NL)
    def _conv_phase():
        p = j % 2
        @pl.when(d == 0)
        def _zero_stats():
            s_scr[p, :, 0:2] = jnp.zeros((C, 2), jnp.float32)

        xb = x_ref[...].astype(jnp.bfloat16)
        st = jnp.concatenate(
            [jnp.roll(xb * mh_ref[0, 0:1, :], W, axis=1), xb,
             jnp.roll(xb * mh_ref[1, 0:1, :], -W, axis=1)],
            axis=0)
        y3 = jnp.dot(wf_ref[...], st, preferred_element_type=jnp.float32)
        y = (y3[C:2 * C] + jnp.roll(y3[0:C] * mf_ref[0, 0:1, :], 1, axis=1)) + (
            jnp.roll(y3[2 * C:] * mf_ref[1, 0:1, :], -1, axis=1)
            + p_ref[NL * i + j, :, 0:1])

        ones = ones_ref[...]
        s_scr[p, :, 0:1] = s_scr[p, :, 0:1] + jnp.dot(
            y, ones, preferred_element_type=jnp.float32)
        s_scr[p, :, 1:2] = s_scr[p, :, 1:2] + jnp.dot(
            y * y, ones, preferred_element_type=jnp.float32)

        y_scr[p, :, pl.ds(d * L, L)] = y.astype(jnp.bfloat16)

    @pl.when(j > 0)
    def _apply_phase():
        q = (j + 1) % 2
        n_a = NL * i + j - 1
        @pl.when(d == 0)
        def _finalize():
            ssum = s_scr[q, :, 0:1]
            ssq = s_scr[q, :, 1:2]
            cper = C // G
            ri = jax.lax.broadcasted_iota(jnp.int32, (C, C), 0) // cper
            ci = jax.lax.broadcasted_iota(jnp.int32, (C, C), 1) // cper
            onehot = (ri == ci).astype(jnp.float32)
            gsum = jnp.dot(onehot, ssum, precision=jax.lax.Precision.HIGHEST)
            gsq = jnp.dot(onehot, ssq, precision=jax.lax.Precision.HIGHEST)
            mean = gsum * inv_cnt
            var = jnp.maximum(gsq * inv_cnt - mean * mean, 0.0)
            rstd = jax.lax.rsqrt(var + eps)
            a = rstd * p_ref[n_a, :, 1:2]
            s_scr[q, :, 2:3] = a
            s_scr[q, :, 3:4] = p_ref[n_a, :, 2:3] - mean * a

        a = s_scr[q, :, 2:3]
        b = s_scr[q, :, 3:4]
        z = y_scr[q, :, pl.ds(d * L, L)].astype(jnp.float32) * a + b
        o_ref[...] = z * jax.nn.sigmoid(z)


def kernel(x, w, bias, gamma, beta, scale, shift):
    N, C_in, D, H, W = x.shape
    C = w.shape[-1]
    HW = H * W
    DHW = D * HW
    G = 8
    eps = 1e-5
    inv_cnt = 1.0 / (DHW * (C // G))

    xr = x.reshape(N, C_in, DHW)
    wf = jnp.transpose(w, (1, 3, 0, 2)).reshape(3 * C, 3 * C_in)
    wf = wf.astype(jnp.bfloat16)
    sc1 = scale + 1.0
    g1 = gamma[None, :] * sc1
    g2 = beta[None, :] * sc1 + shift
    p_all = jnp.stack(
        [jnp.broadcast_to(bias.reshape(1, C), (N, C)), g1, g2], axis=-1)

    DB = 8
    L = DB * HW
    ND = D // DB
    CORES = 2
    NL = N // CORES
    lane = jnp.arange(L, dtype=jnp.int32)
    m31 = (lane % W != W - 1).astype(jnp.float32)
    m0 = (lane % W != 0).astype(jnp.float32)
    mf = jnp.broadcast_to(jnp.stack([m31, m0])[:, None, :], (2, 8, L))
    hrow = (lane // W) % H
    mh31 = (hrow != H - 1).astype(jnp.bfloat16)
    mh0 = (hrow != 0).astype(jnp.bfloat16)
    mh = jnp.broadcast_to(jnp.stack([mh31, mh0])[:, None, :], (2, 8, L))
    ones_col = jnp.ones((L, 1), jnp.float32)

    grid = (CORES, NL + 1, ND)
    kern = functools.partial(_fused_kernel, W=W, C_in=C_in, C=C, DB=DB,
                             HW=HW, G=G, NL=NL, ND=ND, inv_cnt=inv_cnt,
                             eps=eps)
    out = pl.pallas_call(
        kern,
        out_shape=jax.ShapeDtypeStruct((N, C, DHW), jnp.float32),
        grid=grid,
        in_specs=[
            pl.BlockSpec(
                (None, C_in, L),
                lambda i, j, d: (NL * i + jnp.minimum(j, NL - 1), 0,
                                 jnp.where(j == NL, ND - 1, d))),
            pl.BlockSpec((3 * C, 3 * C_in), lambda i, j, d: (0, 0)),
            pl.BlockSpec((N, C, 3), lambda i, j, d: (0, 0, 0)),
            pl.BlockSpec((2, 8, L), lambda i, j, d: (0, 0, 0)),
            pl.BlockSpec((2, 8, L), lambda i, j, d: (0, 0, 0)),
            pl.BlockSpec((L, 1), lambda i, j, d: (0, 0)),
        ],
        out_specs=pl.BlockSpec(
            (None, C, L),
            lambda i, j, d: (NL * i + jnp.maximum(j, 1) - 1, 0, d)),
        scratch_shapes=[
            pltpu.VMEM((2, C, DHW), jnp.bfloat16),
            pltpu.VMEM((2, C, 128), jnp.float32),
        ],
        compiler_params=pltpu.CompilerParams(
            dimension_semantics=("parallel", "arbitrary", "arbitrary")),
    )(xr, wf, p_all, mf, mh, ones_col)
    return out.reshape(N, C, D, H, W)
